# baseline (speedup 1.0000x reference)
import functools

import jax
import jax.numpy as jnp
from jax.experimental import pallas as pl
from jax.experimental.pallas import tpu as pltpu
from jax.experimental.pallas import tpu_sc as plsc

_V = 100000
_EPS = 0.1
_CONF = 1.0 - _EPS
_SMOOTH = _EPS / (_V - 2)
_IGNORE = 0

_LANES = 128
_VA = _V // _LANES * _LANES
_CMAX = (_V - 160) // _LANES * _LANES
_TAIL = _CMAX + _LANES
_BR = 32

_SCL = 16
_NVREG = _V // _SCL
_ROWS_SC = 512
_N_TC = 1024 - _ROWS_SC
_SC_PER_STEP = _ROWS_SC // (_N_TC // _BR)


def _tc_kernel(t_sm, x_ref, pred_any, t_ref, out_ref, scout_ref,
               sliver_ref, sliver2_ref, sem):
    bi = pl.program_id(0)
    base = bi * _BR
    base_sc = _N_TC + bi * _SC_PER_STEP

    def _sliver_copy(local):
        r = base + local
        c = jnp.minimum((t_sm[r] // _LANES) * _LANES, _CMAX)
        return pltpu.make_async_copy(
            pred_any.at[r, pl.ds(c, _LANES)], sliver_ref.at[local], sem)

    def _sliver2_copy(local):
        r = base_sc + local
        c = jnp.minimum((t_sm[r] // _LANES) * _LANES, _CMAX)
        return pltpu.make_async_copy(
            pred_any.at[r, pl.ds(c, _LANES)], sliver2_ref.at[local], sem)

    def _issue(local, carry):
        _sliver_copy(local).start()
        return carry

    def _issue2(local, carry):
        _sliver2_copy(local).start()
        return carry

    jax.lax.fori_loop(0, _BR, _issue, 0)
    jax.lax.fori_loop(0, _SC_PER_STEP, _issue2, 0)

    x = x_ref[...]
    xa = x[:, :_VA]
    xt = x[:, _VA:_V]
    m = jnp.maximum(jnp.max(xa, axis=1, keepdims=True),
                    jnp.max(xt, axis=1, keepdims=True))
    rowsum = (jnp.sum(xa, axis=1, keepdims=True)
              + jnp.sum(xt, axis=1, keepdims=True))
    es = (jnp.sum(jnp.exp(xa - m), axis=1, keepdims=True)
          + jnp.sum(jnp.exp(xt - m), axis=1, keepdims=True))
    p0 = x[:, 0:1]

    t = t_ref[...]
    tail_ids = _VA + jax.lax.broadcasted_iota(jnp.int32, (_BR, _V - _VA), 1)
    pt_tail = jnp.sum(jnp.where(tail_ids == t, xt, 0.0), axis=1,
                      keepdims=True)

    def _wait(local, carry):
        _sliver_copy(local).wait()
        return carry

    def _wait2(local, carry):
        _sliver2_copy(local).wait()
        return carry

    jax.lax.fori_loop(0, _BR, _wait, 0)
    jax.lax.fori_loop(0, _SC_PER_STEP, _wait2, 0)
    scout_ref[...] = sliver2_ref[...]

    g = sliver_ref[...]
    c_vec = jnp.minimum((t // _LANES) * _LANES, _CMAX)
    lane = t - c_vec
    lane_ids = jax.lax.broadcasted_iota(jnp.int32, (_BR, _LANES), 1)
    pt_sliver = jnp.sum(jnp.where(lane_ids == lane, g, 0.0), axis=1,
                        keepdims=True)
    pt = jnp.where(t >= _TAIL, pt_tail, pt_sliver)

    lse = m + jnp.log(es)
    logp_t = pt - lse
    logp_0 = p0 - lse
    s_row = rowsum - jnp.float32(_V) * lse
    contrib = _SMOOTH * (s_row - logp_t - logp_0) + _CONF * logp_t
    rmask = t != _IGNORE
    contrib = jnp.where(rmask, contrib, 0.0)
    csum = jnp.sum(contrib).reshape(1, 1)
    nv = jnp.sum(rmask.astype(jnp.float32)).reshape(1, 1)
    out_ref[...] = jnp.concatenate([csum, nv], axis=1).reshape(1, 1, 2)


def _sc_stats(pred2):
    info = plsc.get_sparse_core_info()
    nw = info.num_cores * info.num_subcores
    nr = _ROWS_SC // nw
    mesh = plsc.VectorSubcoreMesh(core_axis_name="c", subcore_axis_name="s")
    stat = jax.ShapeDtypeStruct((_ROWS_SC * _SCL,), jnp.float32)

    @functools.partial(
        pl.kernel,
        mesh=mesh,
        out_type=[stat, stat, stat, stat],
        scratch_types=[
            pltpu.VMEM((_V,), jnp.float32),
            pltpu.VMEM((nr * _SCL,), jnp.float32),
            pltpu.VMEM((nr * _SCL,), jnp.float32),
            pltpu.VMEM((nr * _SCL,), jnp.float32),
            pltpu.VMEM((nr * _SCL,), jnp.float32),
        ],
    )
    def k(pred_hbm, m_out, sum_out, es_out, p0_out,
          row_v, m_b, s_b, e_b, p0_b):
        wid = jax.lax.axis_index("s") * info.num_cores + jax.lax.axis_index("c")
        base = wid * nr

        def row_body(l, carry):
            row = _N_TC + base + l
            pltpu.sync_copy(pred_hbm.at[row], row_v)

            def p1(i, c):
                mv, sv = c
                v = row_v[pl.ds(i * _SCL, _SCL)]
                return jnp.maximum(mv, v), sv + v

            mv, sv = jax.lax.fori_loop(
                0, _NVREG, p1,
                (jnp.full((_SCL,), -jnp.inf, jnp.float32),
                 jnp.zeros((_SCL,), jnp.float32)))

            def p2(i, ev):
                v = row_v[pl.ds(i * _SCL, _SCL)]
                return ev + jnp.exp(v - mv)

            ev = jax.lax.fori_loop(0, _NVREG, p2,
                                   jnp.zeros((_SCL,), jnp.float32))

            m_b[pl.ds(l * _SCL, _SCL)] = mv
            s_b[pl.ds(l * _SCL, _SCL)] = sv
            e_b[pl.ds(l * _SCL, _SCL)] = ev
            p0_b[pl.ds(l * _SCL, _SCL)] = row_v[pl.ds(0, _SCL)]
            return carry

        jax.lax.fori_loop(0, nr, row_body, 0)

        obase = base * _SCL
        pltpu.sync_copy(m_b, m_out.at[pl.ds(obase, nr * _SCL)])
        pltpu.sync_copy(s_b, sum_out.at[pl.ds(obase, nr * _SCL)])
        pltpu.sync_copy(e_b, es_out.at[pl.ds(obase, nr * _SCL)])
        pltpu.sync_copy(p0_b, p0_out.at[pl.ds(obase, nr * _SCL)])

    return k(pred2)


def _combine_kernel(p_ref, m_ref, sum_ref, es_ref, p0_ref, sliv_ref,
                    xt_ref, tsc_ref, out_ref):
    p = p_ref[...].reshape(-1, 2)
    csum_tc = jnp.sum(p[:, 0:1])
    nv_tc = jnp.sum(p[:, 1:2])

    m2 = m_ref[...]
    m_row = jnp.max(m2, axis=1, keepdims=True)
    es_row = jnp.sum(es_ref[...] * jnp.exp(m2 - m_row), axis=1, keepdims=True)
    sum_row = jnp.sum(sum_ref[...], axis=1, keepdims=True)
    p0_row = p0_ref[...][:, 0:1]

    t = tsc_ref[...]
    g = sliv_ref[...]
    c_vec = jnp.minimum((t // _LANES) * _LANES, _CMAX)
    lane = t - c_vec
    lane_ids = jax.lax.broadcasted_iota(jnp.int32, (_ROWS_SC, _LANES), 1)
    pt_sliver = jnp.sum(jnp.where(lane_ids == lane, g, 0.0), axis=1,
                        keepdims=True)
    xt = xt_ref[...]
    tail_ids = _VA + jax.lax.broadcasted_iota(jnp.int32, (_ROWS_SC, _V - _VA),
                                              1)
    pt_tail = jnp.sum(jnp.where(tail_ids == t, xt, 0.0), axis=1,
                      keepdims=True)
    pt_row = jnp.where(t >= _TAIL, pt_tail, pt_sliver)

    lse = m_row + jnp.log(es_row)
    logp_t = pt_row - lse
    logp_0 = p0_row - lse
    s_row = sum_row - jnp.float32(_V) * lse
    contrib = _SMOOTH * (s_row - logp_t - logp_0) + _CONF * logp_t
    rmask = t != _IGNORE
    contrib = jnp.where(rmask, contrib, 0.0)
    csum = csum_tc + jnp.sum(contrib)
    nv = nv_tc + jnp.sum(rmask.astype(jnp.float32))
    out_ref[...] = (-csum / jnp.maximum(nv, 1.0)).reshape(1, 1)


def kernel(pred, target):
    pred2 = pred.reshape(-1, pred.shape[-1])
    n = pred2.shape[0]
    t = target.reshape(n).astype(jnp.int32)
    nb = _N_TC // _BR

    m_sc, sum_sc, es_sc, p0_sc = _sc_stats(pred2)

    grid_spec = pltpu.PrefetchScalarGridSpec(
        num_scalar_prefetch=1,
        grid=(nb,),
        in_specs=[
            pl.BlockSpec((_BR, _V), lambda b, t_sm: (b, 0)),
            pl.BlockSpec(memory_space=pltpu.MemorySpace.HBM),
            pl.BlockSpec((_BR, 1), lambda b, t_sm: (b, 0)),
        ],
        out_specs=[
            pl.BlockSpec((1, 1, 2), lambda b, t_sm: (b, 0, 0)),
            pl.BlockSpec((_SC_PER_STEP, _LANES), lambda b, t_sm: (b, 0)),
        ],
        scratch_shapes=[
            pltpu.VMEM((_BR, _LANES), jnp.float32),
            pltpu.VMEM((_SC_PER_STEP, _LANES), jnp.float32),
            pltpu.SemaphoreType.DMA,
        ],
    )
    partials, slivers_sc = pl.pallas_call(
        _tc_kernel,
        grid_spec=grid_spec,
        out_shape=[
            jax.ShapeDtypeStruct((nb, 1, 2), jnp.float32),
            jax.ShapeDtypeStruct((_ROWS_SC, _LANES), jnp.float32),
        ],
    )(t, pred2, pred2, t.reshape(n, 1))

    out = pl.pallas_call(
        _combine_kernel,
        out_shape=jax.ShapeDtypeStruct((1, 1), jnp.float32),
    )(partials,
      m_sc.reshape(_ROWS_SC, _SCL), sum_sc.reshape(_ROWS_SC, _SCL),
      es_sc.reshape(_ROWS_SC, _SCL), p0_sc.reshape(_ROWS_SC, _SCL),
      slivers_sc, pred2[_N_TC:, _VA:], t[_N_TC:].reshape(_ROWS_SC, 1))
    return out[0, 0]


# speedup vs baseline: 1.7920x; 1.7920x over previous
import functools

import jax
import jax.numpy as jnp
from jax.experimental import pallas as pl
from jax.experimental.pallas import tpu as pltpu
from jax.experimental.pallas import tpu_sc as plsc

_V = 100000
_EPS = 0.1
_CONF = 1.0 - _EPS
_SMOOTH = _EPS / (_V - 2)
_IGNORE = 0

_LANES = 128
_VA = _V // _LANES * _LANES
_CMAX = (_V - 160) // _LANES * _LANES
_TAIL = _CMAX + _LANES
_BR = 32

_SCL = 16
_NVREG = _V // _SCL
_UNROLL = 25
_NITER = _NVREG // _UNROLL
_ROWS_SC = 512
_N_TC = 1024 - _ROWS_SC
_SC_PER_STEP = _ROWS_SC // (_N_TC // _BR)


def _tc_kernel(t_sm, x_ref, pred_any, t_ref, out_ref, scout_ref,
               sliver_ref, sliver2_ref, sem):
    bi = pl.program_id(0)
    base = bi * _BR
    base_sc = _N_TC + bi * _SC_PER_STEP

    def _sliver_copy(local):
        r = base + local
        c = jnp.minimum((t_sm[r] // _LANES) * _LANES, _CMAX)
        return pltpu.make_async_copy(
            pred_any.at[r, pl.ds(c, _LANES)], sliver_ref.at[local], sem)

    def _sliver2_copy(local):
        r = base_sc + local
        c = jnp.minimum((t_sm[r] // _LANES) * _LANES, _CMAX)
        return pltpu.make_async_copy(
            pred_any.at[r, pl.ds(c, _LANES)], sliver2_ref.at[local], sem)

    def _issue(local, carry):
        _sliver_copy(local).start()
        return carry

    def _issue2(local, carry):
        _sliver2_copy(local).start()
        return carry

    jax.lax.fori_loop(0, _BR, _issue, 0)
    jax.lax.fori_loop(0, _SC_PER_STEP, _issue2, 0)

    x = x_ref[...]
    xa = x[:, :_VA]
    xt = x[:, _VA:_V]
    m = jnp.maximum(jnp.max(xa, axis=1, keepdims=True),
                    jnp.max(xt, axis=1, keepdims=True))
    rowsum = (jnp.sum(xa, axis=1, keepdims=True)
              + jnp.sum(xt, axis=1, keepdims=True))
    es = (jnp.sum(jnp.exp(xa - m), axis=1, keepdims=True)
          + jnp.sum(jnp.exp(xt - m), axis=1, keepdims=True))
    p0 = x[:, 0:1]

    t = t_ref[...]
    tail_ids = _VA + jax.lax.broadcasted_iota(jnp.int32, (_BR, _V - _VA), 1)
    pt_tail = jnp.sum(jnp.where(tail_ids == t, xt, 0.0), axis=1,
                      keepdims=True)

    def _wait(local, carry):
        _sliver_copy(local).wait()
        return carry

    def _wait2(local, carry):
        _sliver2_copy(local).wait()
        return carry

    jax.lax.fori_loop(0, _BR, _wait, 0)
    jax.lax.fori_loop(0, _SC_PER_STEP, _wait2, 0)
    scout_ref[...] = sliver2_ref[...]

    g = sliver_ref[...]
    c_vec = jnp.minimum((t // _LANES) * _LANES, _CMAX)
    lane = t - c_vec
    lane_ids = jax.lax.broadcasted_iota(jnp.int32, (_BR, _LANES), 1)
    pt_sliver = jnp.sum(jnp.where(lane_ids == lane, g, 0.0), axis=1,
                        keepdims=True)
    pt = jnp.where(t >= _TAIL, pt_tail, pt_sliver)

    lse = m + jnp.log(es)
    logp_t = pt - lse
    logp_0 = p0 - lse
    s_row = rowsum - jnp.float32(_V) * lse
    contrib = _SMOOTH * (s_row - logp_t - logp_0) + _CONF * logp_t
    rmask = t != _IGNORE
    contrib = jnp.where(rmask, contrib, 0.0)
    csum = jnp.sum(contrib).reshape(1, 1)
    nv = jnp.sum(rmask.astype(jnp.float32)).reshape(1, 1)
    out_ref[...] = jnp.concatenate([csum, nv], axis=1).reshape(1, 1, 2)


def _sc_stats(pred2):
    info = plsc.get_sparse_core_info()
    nw = info.num_cores * info.num_subcores
    nr = _ROWS_SC // nw
    mesh = plsc.VectorSubcoreMesh(core_axis_name="c", subcore_axis_name="s")
    stat = jax.ShapeDtypeStruct((_ROWS_SC * _SCL,), jnp.float32)

    @functools.partial(
        pl.kernel,
        mesh=mesh,
        out_type=[stat, stat, stat, stat],
        scratch_types=[
            pltpu.VMEM((_V,), jnp.float32),
            pltpu.VMEM((nr * _SCL,), jnp.float32),
            pltpu.VMEM((nr * _SCL,), jnp.float32),
            pltpu.VMEM((nr * _SCL,), jnp.float32),
            pltpu.VMEM((nr * _SCL,), jnp.float32),
        ],
    )
    def k(pred_hbm, m_out, sum_out, es_out, p0_out,
          row_v, m_b, s_b, e_b, p0_b):
        wid = jax.lax.axis_index("s") * info.num_cores + jax.lax.axis_index("c")
        base = wid * nr

        def row_body(l, carry):
            row = _N_TC + base + l
            pltpu.sync_copy(pred_hbm.at[row], row_v)

            def p1(i, c):
                mv, sv = c
                b = i * (_UNROLL * _SCL)
                for u in range(_UNROLL):
                    v = row_v[pl.ds(b + u * _SCL, _SCL)]
                    mv = jnp.maximum(mv, v)
                    sv = sv + v
                return mv, sv

            mv, sv = jax.lax.fori_loop(
                0, _NITER, p1,
                (jnp.full((_SCL,), -jnp.inf, jnp.float32),
                 jnp.zeros((_SCL,), jnp.float32)))

            def p2(i, ev):
                b = i * (_UNROLL * _SCL)
                for u in range(_UNROLL):
                    v = row_v[pl.ds(b + u * _SCL, _SCL)]
                    ev = ev + jnp.exp(v - mv)
                return ev

            ev = jax.lax.fori_loop(0, _NITER, p2,
                                   jnp.zeros((_SCL,), jnp.float32))

            m_b[pl.ds(l * _SCL, _SCL)] = mv
            s_b[pl.ds(l * _SCL, _SCL)] = sv
            e_b[pl.ds(l * _SCL, _SCL)] = ev
            p0_b[pl.ds(l * _SCL, _SCL)] = row_v[pl.ds(0, _SCL)]
            return carry

        jax.lax.fori_loop(0, nr, row_body, 0)

        obase = base * _SCL
        pltpu.sync_copy(m_b, m_out.at[pl.ds(obase, nr * _SCL)])
        pltpu.sync_copy(s_b, sum_out.at[pl.ds(obase, nr * _SCL)])
        pltpu.sync_copy(e_b, es_out.at[pl.ds(obase, nr * _SCL)])
        pltpu.sync_copy(p0_b, p0_out.at[pl.ds(obase, nr * _SCL)])

    return k(pred2)


def _combine_kernel(p_ref, m_ref, sum_ref, es_ref, p0_ref, sliv_ref,
                    xt_ref, tsc_ref, out_ref):
    p = p_ref[...].reshape(-1, 2)
    csum_tc = jnp.sum(p[:, 0:1])
    nv_tc = jnp.sum(p[:, 1:2])

    m2 = m_ref[...]
    m_row = jnp.max(m2, axis=1, keepdims=True)
    es_row = jnp.sum(es_ref[...] * jnp.exp(m2 - m_row), axis=1, keepdims=True)
    sum_row = jnp.sum(sum_ref[...], axis=1, keepdims=True)
    p0_row = p0_ref[...][:, 0:1]

    t = tsc_ref[...]
    g = sliv_ref[...]
    c_vec = jnp.minimum((t // _LANES) * _LANES, _CMAX)
    lane = t - c_vec
    lane_ids = jax.lax.broadcasted_iota(jnp.int32, (_ROWS_SC, _LANES), 1)
    pt_sliver = jnp.sum(jnp.where(lane_ids == lane, g, 0.0), axis=1,
                        keepdims=True)
    xt = xt_ref[...]
    tail_ids = _VA + jax.lax.broadcasted_iota(jnp.int32, (_ROWS_SC, _V - _VA),
                                              1)
    pt_tail = jnp.sum(jnp.where(tail_ids == t, xt, 0.0), axis=1,
                      keepdims=True)
    pt_row = jnp.where(t >= _TAIL, pt_tail, pt_sliver)

    lse = m_row + jnp.log(es_row)
    logp_t = pt_row - lse
    logp_0 = p0_row - lse
    s_row = sum_row - jnp.float32(_V) * lse
    contrib = _SMOOTH * (s_row - logp_t - logp_0) + _CONF * logp_t
    rmask = t != _IGNORE
    contrib = jnp.where(rmask, contrib, 0.0)
    csum = csum_tc + jnp.sum(contrib)
    nv = nv_tc + jnp.sum(rmask.astype(jnp.float32))
    out_ref[...] = (-csum / jnp.maximum(nv, 1.0)).reshape(1, 1)


def kernel(pred, target):
    pred2 = pred.reshape(-1, pred.shape[-1])
    n = pred2.shape[0]
    t = target.reshape(n).astype(jnp.int32)
    nb = _N_TC // _BR

    m_sc, sum_sc, es_sc, p0_sc = _sc_stats(pred2)

    grid_spec = pltpu.PrefetchScalarGridSpec(
        num_scalar_prefetch=1,
        grid=(nb,),
        in_specs=[
            pl.BlockSpec((_BR, _V), lambda b, t_sm: (b, 0)),
            pl.BlockSpec(memory_space=pltpu.MemorySpace.HBM),
            pl.BlockSpec((_BR, 1), lambda b, t_sm: (b, 0)),
        ],
        out_specs=[
            pl.BlockSpec((1, 1, 2), lambda b, t_sm: (b, 0, 0)),
            pl.BlockSpec((_SC_PER_STEP, _LANES), lambda b, t_sm: (b, 0)),
        ],
        scratch_shapes=[
            pltpu.VMEM((_BR, _LANES), jnp.float32),
            pltpu.VMEM((_SC_PER_STEP, _LANES), jnp.float32),
            pltpu.SemaphoreType.DMA,
        ],
    )
    partials, slivers_sc = pl.pallas_call(
        _tc_kernel,
        grid_spec=grid_spec,
        out_shape=[
            jax.ShapeDtypeStruct((nb, 1, 2), jnp.float32),
            jax.ShapeDtypeStruct((_ROWS_SC, _LANES), jnp.float32),
        ],
    )(t, pred2, pred2, t.reshape(n, 1))

    out = pl.pallas_call(
        _combine_kernel,
        out_shape=jax.ShapeDtypeStruct((1, 1), jnp.float32),
    )(partials,
      m_sc.reshape(_ROWS_SC, _SCL), sum_sc.reshape(_ROWS_SC, _SCL),
      es_sc.reshape(_ROWS_SC, _SCL), p0_sc.reshape(_ROWS_SC, _SCL),
      slivers_sc, pred2[_N_TC:, _VA:], t[_N_TC:].reshape(_ROWS_SC, 1))
    return out[0, 0]


# speedup vs baseline: 2.2828x; 1.2739x over previous
import functools

import jax
import jax.numpy as jnp
from jax.experimental import pallas as pl
from jax.experimental.pallas import tpu as pltpu
from jax.experimental.pallas import tpu_sc as plsc

_V = 100000
_EPS = 0.1
_CONF = 1.0 - _EPS
_SMOOTH = _EPS / (_V - 2)
_IGNORE = 0

_LANES = 128
_VA = _V // _LANES * _LANES
_CMAX = (_V - 160) // _LANES * _LANES
_TAIL = _CMAX + _LANES
_BR = 32

_SCL = 16
_NVREG = _V // _SCL
_UNROLL = 25
_NITER = _NVREG // _UNROLL
_ROWS_SC = 256
_N_TC = 1024 - _ROWS_SC


def _tc_kernel(t_sm, x_ref, pred_any, t_ref, out_ref, scout_ref,
               sliver_ref, sliver2_ref, sem):
    bi = pl.program_id(0)
    base = bi * _BR

    def _sliver_copy(local):
        r = base + local
        c = jnp.minimum((t_sm[r] // _LANES) * _LANES, _CMAX)
        return pltpu.make_async_copy(
            pred_any.at[r, pl.ds(c, _LANES)], sliver_ref.at[local], sem)

    def _sliver2_copy(local):
        r = _N_TC + local
        c = jnp.minimum((t_sm[r] // _LANES) * _LANES, _CMAX)
        return pltpu.make_async_copy(
            pred_any.at[r, pl.ds(c, _LANES)], sliver2_ref.at[local], sem)

    def _issue(local, carry):
        _sliver_copy(local).start()
        return carry

    def _issue2(local, carry):
        _sliver2_copy(local).start()
        return carry

    jax.lax.fori_loop(0, _BR, _issue, 0)

    @pl.when(bi == 0)
    def _():
        jax.lax.fori_loop(0, _ROWS_SC, _issue2, 0)

    x = x_ref[...]
    xa = x[:, :_VA]
    xt = x[:, _VA:_V]
    m = jnp.maximum(jnp.max(xa, axis=1, keepdims=True),
                    jnp.max(xt, axis=1, keepdims=True))
    rowsum = (jnp.sum(xa, axis=1, keepdims=True)
              + jnp.sum(xt, axis=1, keepdims=True))
    es = (jnp.sum(jnp.exp(xa - m), axis=1, keepdims=True)
          + jnp.sum(jnp.exp(xt - m), axis=1, keepdims=True))
    p0 = x[:, 0:1]

    t = t_ref[...]
    tail_ids = _VA + jax.lax.broadcasted_iota(jnp.int32, (_BR, _V - _VA), 1)
    pt_tail = jnp.sum(jnp.where(tail_ids == t, xt, 0.0), axis=1,
                      keepdims=True)

    def _wait(local, carry):
        _sliver_copy(local).wait()
        return carry

    def _wait2(local, carry):
        _sliver2_copy(local).wait()
        return carry

    jax.lax.fori_loop(0, _BR, _wait, 0)

    @pl.when(bi == 0)
    def _():
        jax.lax.fori_loop(0, _ROWS_SC, _wait2, 0)
        scout_ref[...] = sliver2_ref[...]

    g = sliver_ref[...]
    c_vec = jnp.minimum((t // _LANES) * _LANES, _CMAX)
    lane = t - c_vec
    lane_ids = jax.lax.broadcasted_iota(jnp.int32, (_BR, _LANES), 1)
    pt_sliver = jnp.sum(jnp.where(lane_ids == lane, g, 0.0), axis=1,
                        keepdims=True)
    pt = jnp.where(t >= _TAIL, pt_tail, pt_sliver)

    lse = m + jnp.log(es)
    logp_t = pt - lse
    logp_0 = p0 - lse
    s_row = rowsum - jnp.float32(_V) * lse
    contrib = _SMOOTH * (s_row - logp_t - logp_0) + _CONF * logp_t
    rmask = t != _IGNORE
    contrib = jnp.where(rmask, contrib, 0.0)
    csum = jnp.sum(contrib).reshape(1, 1)
    nv = jnp.sum(rmask.astype(jnp.float32)).reshape(1, 1)
    out_ref[...] = jnp.concatenate([csum, nv], axis=1).reshape(1, 1, 2)


def _sc_stats(pred2):
    info = plsc.get_sparse_core_info()
    nw = info.num_cores * info.num_subcores
    nr = _ROWS_SC // nw
    mesh = plsc.VectorSubcoreMesh(core_axis_name="c", subcore_axis_name="s")
    stat = jax.ShapeDtypeStruct((_ROWS_SC * _SCL,), jnp.float32)

    @functools.partial(
        pl.kernel,
        mesh=mesh,
        out_type=[stat, stat, stat, stat],
        scratch_types=[
            pltpu.VMEM((_V,), jnp.float32),
            pltpu.VMEM((nr * _SCL,), jnp.float32),
            pltpu.VMEM((nr * _SCL,), jnp.float32),
            pltpu.VMEM((nr * _SCL,), jnp.float32),
            pltpu.VMEM((nr * _SCL,), jnp.float32),
        ],
    )
    def k(pred_hbm, m_out, sum_out, es_out, p0_out,
          row_v, m_b, s_b, e_b, p0_b):
        wid = jax.lax.axis_index("s") * info.num_cores + jax.lax.axis_index("c")
        base = wid * nr

        def row_body(l, carry):
            row = _N_TC + base + l
            pltpu.sync_copy(pred_hbm.at[row], row_v)

            def p1(i, c):
                mv, sv = c
                b = i * (_UNROLL * _SCL)
                for u in range(_UNROLL):
                    v = row_v[pl.ds(b + u * _SCL, _SCL)]
                    mv = jnp.maximum(mv, v)
                    sv = sv + v
                return mv, sv

            mv, sv = jax.lax.fori_loop(
                0, _NITER, p1,
                (jnp.full((_SCL,), -jnp.inf, jnp.float32),
                 jnp.zeros((_SCL,), jnp.float32)))

            def p2(i, ev):
                b = i * (_UNROLL * _SCL)
                for u in range(_UNROLL):
                    v = row_v[pl.ds(b + u * _SCL, _SCL)]
                    ev = ev + jnp.exp(v - mv)
                return ev

            ev = jax.lax.fori_loop(0, _NITER, p2,
                                   jnp.zeros((_SCL,), jnp.float32))

            m_b[pl.ds(l * _SCL, _SCL)] = mv
            s_b[pl.ds(l * _SCL, _SCL)] = sv
            e_b[pl.ds(l * _SCL, _SCL)] = ev
            p0_b[pl.ds(l * _SCL, _SCL)] = row_v[pl.ds(0, _SCL)]
            return carry

        jax.lax.fori_loop(0, nr, row_body, 0)

        obase = base * _SCL
        pltpu.sync_copy(m_b, m_out.at[pl.ds(obase, nr * _SCL)])
        pltpu.sync_copy(s_b, sum_out.at[pl.ds(obase, nr * _SCL)])
        pltpu.sync_copy(e_b, es_out.at[pl.ds(obase, nr * _SCL)])
        pltpu.sync_copy(p0_b, p0_out.at[pl.ds(obase, nr * _SCL)])

    return k(pred2)


def _combine_kernel(p_ref, m_ref, sum_ref, es_ref, p0_ref, sliv_ref,
                    xt_ref, tsc_ref, out_ref):
    p = p_ref[...].reshape(-1, 2)
    csum_tc = jnp.sum(p[:, 0:1])
    nv_tc = jnp.sum(p[:, 1:2])

    m2 = m_ref[...]
    m_row = jnp.max(m2, axis=1, keepdims=True)
    es_row = jnp.sum(es_ref[...] * jnp.exp(m2 - m_row), axis=1, keepdims=True)
    sum_row = jnp.sum(sum_ref[...], axis=1, keepdims=True)
    p0_row = p0_ref[...][:, 0:1]

    t = tsc_ref[...]
    g = sliv_ref[...]
    c_vec = jnp.minimum((t // _LANES) * _LANES, _CMAX)
    lane = t - c_vec
    lane_ids = jax.lax.broadcasted_iota(jnp.int32, (_ROWS_SC, _LANES), 1)
    pt_sliver = jnp.sum(jnp.where(lane_ids == lane, g, 0.0), axis=1,
                        keepdims=True)
    xt = xt_ref[...]
    tail_ids = _VA + jax.lax.broadcasted_iota(jnp.int32, (_ROWS_SC, _V - _VA),
                                              1)
    pt_tail = jnp.sum(jnp.where(tail_ids == t, xt, 0.0), axis=1,
                      keepdims=True)
    pt_row = jnp.where(t >= _TAIL, pt_tail, pt_sliver)

    lse = m_row + jnp.log(es_row)
    logp_t = pt_row - lse
    logp_0 = p0_row - lse
    s_row = sum_row - jnp.float32(_V) * lse
    contrib = _SMOOTH * (s_row - logp_t - logp_0) + _CONF * logp_t
    rmask = t != _IGNORE
    contrib = jnp.where(rmask, contrib, 0.0)
    csum = csum_tc + jnp.sum(contrib)
    nv = nv_tc + jnp.sum(rmask.astype(jnp.float32))
    out_ref[...] = (-csum / jnp.maximum(nv, 1.0)).reshape(1, 1)


def kernel(pred, target):
    pred2 = pred.reshape(-1, pred.shape[-1])
    n = pred2.shape[0]
    t = target.reshape(n).astype(jnp.int32)
    nb = _N_TC // _BR

    m_sc, sum_sc, es_sc, p0_sc = _sc_stats(pred2)

    grid_spec = pltpu.PrefetchScalarGridSpec(
        num_scalar_prefetch=1,
        grid=(nb,),
        in_specs=[
            pl.BlockSpec((_BR, _V), lambda b, t_sm: (b, 0)),
            pl.BlockSpec(memory_space=pltpu.MemorySpace.HBM),
            pl.BlockSpec((_BR, 1), lambda b, t_sm: (b, 0)),
        ],
        out_specs=[
            pl.BlockSpec((1, 1, 2), lambda b, t_sm: (b, 0, 0)),
            pl.BlockSpec((_ROWS_SC, _LANES), lambda b, t_sm: (0, 0)),
        ],
        scratch_shapes=[
            pltpu.VMEM((_BR, _LANES), jnp.float32),
            pltpu.VMEM((_ROWS_SC, _LANES), jnp.float32),
            pltpu.SemaphoreType.DMA,
        ],
    )
    partials, slivers_sc = pl.pallas_call(
        _tc_kernel,
        grid_spec=grid_spec,
        out_shape=[
            jax.ShapeDtypeStruct((nb, 1, 2), jnp.float32),
            jax.ShapeDtypeStruct((_ROWS_SC, _LANES), jnp.float32),
        ],
    )(t, pred2, pred2, t.reshape(n, 1))

    out = pl.pallas_call(
        _combine_kernel,
        out_shape=jax.ShapeDtypeStruct((1, 1), jnp.float32),
    )(partials,
      m_sc.reshape(_ROWS_SC, _SCL), sum_sc.reshape(_ROWS_SC, _SCL),
      es_sc.reshape(_ROWS_SC, _SCL), p0_sc.reshape(_ROWS_SC, _SCL),
      slivers_sc, pred2[_N_TC:, _VA:], t[_N_TC:].reshape(_ROWS_SC, 1))
    return out[0, 0]
